# baseline (speedup 1.0000x reference)
import functools
import math

import jax
import jax.numpy as jnp
from jax import lax
from jax.experimental import pallas as pl
from jax.experimental.pallas import tpu as pltpu
from jax.experimental.pallas import tpu_sc as plsc

EPS = 1e-5
_NEG = -3.4e38
BT = 256
CAP = 36864
NBLK = CAP // BT


def _normalize(x):
    m = jnp.mean(x, axis=-1, keepdims=True)
    v = jnp.mean((x - m) ** 2, axis=-1, keepdims=True)
    return (x - m) * jax.lax.rsqrt(v + EPS)


def _gelu(x):
    return x * 0.5 * (1.0 + jax.lax.erf(x * (1.0 / math.sqrt(2.0))))


def _router_block(ln_in_g_r, ln_in_b_r, Wr_r, cr_r, x_r,
                  znx_r, logits_r, i1_r, i2_r, w1_r, w2_r,
                  frac_r, prob_r, cnt_r):
    i = pl.program_id(0)
    nsteps = pl.num_programs(0)
    E = Wr_r.shape[1]

    x = x_r[...]
    xn = _normalize(x) * ln_in_g_r[...] + ln_in_b_r[...]
    znx = _normalize(xn)
    znx_r[...] = znx

    logits = jnp.dot(znx, Wr_r[...], preferred_element_type=jnp.float32)
    logits = logits + cr_r[...]
    logits_r[...] = logits

    iota = jax.lax.broadcasted_iota(jnp.int32, logits.shape, 1)
    v1 = jnp.max(logits, axis=-1, keepdims=True)
    i1 = jnp.min(jnp.where(logits == v1, iota, E), axis=-1, keepdims=True)
    masked = jnp.where(iota == i1, _NEG, logits)
    v2 = jnp.max(masked, axis=-1, keepdims=True)
    i2 = jnp.min(jnp.where(masked == v2, iota, E), axis=-1, keepdims=True)
    t = jnp.exp(v2 - v1)
    w1 = 1.0 / (1.0 + t)
    w2 = t / (1.0 + t)

    i1_r[...] = i1[:, 0][None, None, :]
    i2_r[...] = i2[:, 0][None, None, :]
    w1_r[...] = w1[:, 0][None, None, :]
    w2_r[...] = w2[:, 0][None, None, :]

    routed = ((iota == i1).astype(jnp.float32)
              + (iota == i2).astype(jnp.float32))
    sm = jnp.exp(logits - v1)
    sm = sm / jnp.sum(sm, axis=-1, keepdims=True)

    @pl.when(i == 0)
    def _():
        frac_r[...] = jnp.zeros_like(frac_r)
        prob_r[...] = jnp.zeros_like(prob_r)
        cnt_r[...] = jnp.zeros_like(cnt_r)

    frac_r[...] += jnp.sum(routed, axis=0, keepdims=True)
    prob_r[...] += jnp.sum(sm, axis=0, keepdims=True)
    cnt_r[...] += jnp.sum(routed, axis=0, keepdims=True)

    @pl.when(i == nsteps - 1)
    def _():
        n_total = nsteps * x.shape[0]
        frac_r[...] *= 1.0 / n_total
        prob_r[...] *= 1.0 / n_total


def _pos_block(cnt_r, tril_r, i1_r, i2_r, pos0_r, pos1_r, blk_r,
               carry_r, off_r):
    i = pl.program_id(0)
    B = i1_r.shape[2]
    E = cnt_r.shape[1]

    @pl.when(i == 0)
    def _():
        cnt = cnt_r[...]
        capac = jnp.ceil(cnt * (1.0 / BT)) * float(BT)
        tri = (jax.lax.broadcasted_iota(jnp.int32, (E, E), 0)
               < jax.lax.broadcasted_iota(jnp.int32, (E, E), 1)
               ).astype(jnp.float32)
        off = jnp.dot(capac, tri, preferred_element_type=jnp.float32)
        off_r[...] = off
        carry_r[...] = jnp.zeros_like(carry_r)
        bpos = jax.lax.broadcasted_iota(
            jnp.int32, (1, NBLK), 1).astype(jnp.float32) * float(BT)
        be = jnp.zeros((1, NBLK), jnp.float32)
        for e in range(1, E):
            off_e = off[:, e:e + 1]
            be = be + (bpos >= off_e).astype(jnp.float32)
        blk_r[...] = be.astype(jnp.int32)

    off = off_r[...]
    tril = tril_r[...]
    iota_e = jax.lax.broadcasted_iota(jnp.int32, (B, E), 1)

    for keys_r, pos_r in ((i1_r, pos0_r), (i2_r, pos1_r)):
        keys = keys_r[0, 0, :]
        mask = (keys[:, None] == iota_e).astype(jnp.float32)
        incl = jnp.dot(tril, mask.astype(jnp.bfloat16),
                       preferred_element_type=jnp.float32)
        base = off + carry_r[...]
        slot = jnp.sum(mask * (base + incl - 1.0), axis=-1)
        pos_r[...] = slot.astype(jnp.int32)[None, None, :]
        carry_r[...] += jnp.sum(mask, axis=0, keepdims=True)


def _expert_block(be_r, zp_r, W1_r, c1_r, W2_r, c2_r, W3_r, c3_r, out_r):
    e = be_r[pl.program_id(0)]
    zb = zp_r[...].astype(jnp.bfloat16)
    h1 = jnp.dot(zb, W1_r[e], preferred_element_type=jnp.float32)
    h1 = _gelu(h1 + c1_r[e])
    n1 = _normalize(h1).astype(jnp.bfloat16)
    h2 = jnp.dot(n1, W2_r[e], preferred_element_type=jnp.float32)
    h2 = _gelu(h2 + c2_r[e])
    n2 = _normalize(h2)
    out_e = jnp.sum(n2 * W3_r[e], axis=-1) + c3_r[e, 0]
    out_r[...] = out_e[:, None]


def _make_sc_kernels(N, D):
    mesh = plsc.VectorSubcoreMesh(core_axis_name="c", subcore_axis_name="s")
    NW = 32
    A_PER_W = 2 * N // NW
    S_PER_W = CAP // NW
    T_PER_W = N // NW

    @functools.partial(
        pl.kernel, mesh=mesh,
        out_type=jax.ShapeDtypeStruct((CAP,), jnp.int32),
        scratch_types=[
            pltpu.VMEM((8, 128), jnp.int32),
            pltpu.VMEM((8, 128), jnp.int32),
            pltpu.SemaphoreType.DMA,
        ],
    )
    def k_scatter(poscat, tokperm, posv, tokv, sem):
        wid = lax.axis_index("s") * 2 + lax.axis_index("c")
        base = wid * A_PER_W
        for j in range(8):
            pltpu.sync_copy(poscat.at[pl.ds(base + j * 128, 128)],
                            posv.at[j])
            for i in range(8):
                a16 = (base + j * 128 + i * 16
                       + jax.lax.iota(jnp.int32, 16))
                tokv[j, pl.ds(i * 16, 16)] = a16 & (N - 1)
        copies = []
        for j in range(8):
            copies.append(pltpu.async_copy(
                tokv.at[j], tokperm.at[posv.at[j]], sem))
        for c in copies:
            c.wait()

    @functools.partial(
        pl.kernel, mesh=mesh,
        out_type=jax.ShapeDtypeStruct((CAP, D), jnp.float32),
        scratch_types=[
            pltpu.VMEM((9, 128), jnp.int32),
            pltpu.VMEM((128, D), jnp.float32),
            pltpu.SemaphoreType.DMA,
        ],
    )
    def k_gather(tokperm, znx, znx_perm, idxv, rows, sem):
        wid = lax.axis_index("s") * 2 + lax.axis_index("c")
        base = wid * S_PER_W
        for j in range(9):
            pltpu.sync_copy(tokperm.at[pl.ds(base + j * 128, 128)],
                            idxv.at[j])
        for j in range(9):
            for i in range(8):
                v = idxv[j, pl.ds(i * 16, 16)]
                idxv[j, pl.ds(i * 16, 16)] = v & (N - 1)
        for j in range(9):
            pltpu.async_copy(znx.at[idxv.at[j]], rows, sem).wait()
            pltpu.sync_copy(rows,
                            znx_perm.at[pl.ds(base + j * 128, 128)])

    @functools.partial(
        pl.kernel, mesh=mesh,
        out_type=jax.ShapeDtypeStruct((N,), jnp.float32),
        scratch_types=[
            pltpu.VMEM((4, 128), jnp.int32),
            pltpu.VMEM((4, 128), jnp.int32),
            pltpu.VMEM((4, 128), jnp.float32),
            pltpu.VMEM((4, 128), jnp.float32),
            pltpu.VMEM((4, 128), jnp.float32),
            pltpu.VMEM((4, 128), jnp.float32),
            pltpu.VMEM((4, 128), jnp.float32),
            pltpu.SemaphoreType.DMA,
        ],
    )
    def k_combine(pos0, pos1, w1, w2, outperm, final,
                  p0, p1, wv0, wv1, g0, g1, fin, sem):
        wid = lax.axis_index("s") * 2 + lax.axis_index("c")
        base = wid * T_PER_W
        for j in range(4):
            pltpu.sync_copy(pos0.at[pl.ds(base + j * 128, 128)], p0.at[j])
            pltpu.sync_copy(pos1.at[pl.ds(base + j * 128, 128)], p1.at[j])
            pltpu.sync_copy(w1.at[pl.ds(base + j * 128, 128)], wv0.at[j])
            pltpu.sync_copy(w2.at[pl.ds(base + j * 128, 128)], wv1.at[j])
        for j in range(4):
            pltpu.async_copy(outperm.at[p0.at[j]], g0.at[j], sem).wait()
            pltpu.async_copy(outperm.at[p1.at[j]], g1.at[j], sem).wait()
        for j in range(4):
            for r in range(8):
                sl = pl.ds(r * 16, 16)
                fin[j, sl] = g0[j, sl] * wv0[j, sl] + g1[j, sl] * wv1[j, sl]
        for j in range(4):
            pltpu.sync_copy(fin.at[j],
                            final.at[pl.ds(base + j * 128, 128)])

    return k_scatter, k_gather, k_combine


def kernel(x, ln_in_g, ln_in_b, ln_r_g, ln_r_b, W_r, b_r, e_ln1_g, e_ln1_b,
           e_W1, e_b1, e_ln2_g, e_ln2_b, e_W2, e_b2, e_ln3_g, e_ln3_b,
           e_W3, e_b3):
    N, D = x.shape
    E = e_W1.shape[0]
    H = e_W1.shape[2]
    H2 = e_W2.shape[2]
    B = 512
    grid = (N // B,)
    NB = N // B

    Wr_f = ln_r_g[:, None] * W_r
    cr = ln_r_b @ W_r + b_r
    W1_f = (e_ln1_g[:, :, None] * e_W1).astype(jnp.bfloat16)
    c1 = jnp.einsum("ed,edh->eh", e_ln1_b, e_W1) + e_b1
    W2_f = (e_ln2_g[:, :, None] * e_W2).astype(jnp.bfloat16)
    c2 = jnp.einsum("eh,ehk->ek", e_ln2_b, e_W2) + e_b2
    W3_f = e_ln3_g * e_W3[:, :, 0]
    c3 = (jnp.sum(e_ln3_b * e_W3[:, :, 0], axis=-1, keepdims=True) + e_b3)

    full = lambda *s: pl.BlockSpec(s, lambda i: (0,) * len(s))

    a_out = pl.pallas_call(
        _router_block,
        grid=grid,
        in_specs=[full(1, D), full(1, D), full(D, E), full(1, E),
                  pl.BlockSpec((B, D), lambda i: (i, 0))],
        out_specs=[
            pl.BlockSpec((B, D), lambda i: (i, 0)),
            pl.BlockSpec((B, E), lambda i: (i, 0)),
            pl.BlockSpec((1, 1, B), lambda i: (i, 0, 0)),
            pl.BlockSpec((1, 1, B), lambda i: (i, 0, 0)),
            pl.BlockSpec((1, 1, B), lambda i: (i, 0, 0)),
            pl.BlockSpec((1, 1, B), lambda i: (i, 0, 0)),
            pl.BlockSpec((1, E), lambda i: (0, 0)),
            pl.BlockSpec((1, E), lambda i: (0, 0)),
            pl.BlockSpec((1, E), lambda i: (0, 0)),
        ],
        out_shape=(
            jax.ShapeDtypeStruct((N, D), jnp.float32),
            jax.ShapeDtypeStruct((N, E), jnp.float32),
            jax.ShapeDtypeStruct((NB, 1, B), jnp.int32),
            jax.ShapeDtypeStruct((NB, 1, B), jnp.int32),
            jax.ShapeDtypeStruct((NB, 1, B), jnp.float32),
            jax.ShapeDtypeStruct((NB, 1, B), jnp.float32),
            jax.ShapeDtypeStruct((1, E), jnp.float32),
            jax.ShapeDtypeStruct((1, E), jnp.float32),
            jax.ShapeDtypeStruct((1, E), jnp.float32),
        ),
    )(ln_in_g[None], ln_in_b[None], Wr_f, cr[None], x)
    znx, logits, i1b, i2b, w1b, w2b, frac, prob, cnt = a_out

    tril_const = (jnp.arange(B)[:, None] >= jnp.arange(B)[None, :]
                  ).astype(jnp.bfloat16)
    pos0b, pos1b, blk = pl.pallas_call(
        _pos_block,
        grid=grid,
        in_specs=[full(1, E), full(B, B),
                  pl.BlockSpec((1, 1, B), lambda i: (i, 0, 0)),
                  pl.BlockSpec((1, 1, B), lambda i: (i, 0, 0))],
        out_specs=[
            pl.BlockSpec((1, 1, B), lambda i: (i, 0, 0)),
            pl.BlockSpec((1, 1, B), lambda i: (i, 0, 0)),
            pl.BlockSpec((1, NBLK), lambda i: (0, 0)),
        ],
        out_shape=(
            jax.ShapeDtypeStruct((NB, 1, B), jnp.int32),
            jax.ShapeDtypeStruct((NB, 1, B), jnp.int32),
            jax.ShapeDtypeStruct((1, NBLK), jnp.int32),
        ),
        scratch_shapes=[pltpu.VMEM((1, E), jnp.float32),
                        pltpu.VMEM((1, E), jnp.float32)],
    )(cnt, tril_const, i1b, i2b)

    pos0 = pos0b.reshape(N)
    pos1 = pos1b.reshape(N)
    poscat = jnp.concatenate([pos0, pos1], axis=0)

    k_scatter, k_gather, k_combine = _make_sc_kernels(N, D)
    tokperm = k_scatter(poscat)
    znx_perm = k_gather(tokperm, znx)

    outperm = pl.pallas_call(
        _expert_block,
        grid_spec=pltpu.PrefetchScalarGridSpec(
            num_scalar_prefetch=1,
            grid=(NBLK,),
            in_specs=[
                pl.BlockSpec((BT, D), lambda i, s: (i, 0)),
                pl.BlockSpec((E, D, H), lambda i, s: (0, 0, 0)),
                pl.BlockSpec((E, 1, H), lambda i, s: (0, 0, 0)),
                pl.BlockSpec((E, H, H2), lambda i, s: (0, 0, 0)),
                pl.BlockSpec((E, 1, H2), lambda i, s: (0, 0, 0)),
                pl.BlockSpec((E, 1, H2), lambda i, s: (0, 0, 0)),
                pl.BlockSpec((E, 1), lambda i, s: (0, 0)),
            ],
            out_specs=pl.BlockSpec((BT, 1), lambda i, s: (i, 0)),
        ),
        out_shape=jax.ShapeDtypeStruct((CAP, 1), jnp.float32),
    )(blk.reshape(NBLK), znx_perm, W1_f, c1[:, None], W2_f, c2[:, None],
      W3_f[:, None], c3)

    final = k_combine(pos0, pos1, w1b.reshape(N), w2b.reshape(N),
                      outperm.reshape(CAP))

    return (final[:, None], frac[0], prob[0], logits)


# speedup vs baseline: 1.3783x; 1.3783x over previous
import functools
import math

import jax
import jax.numpy as jnp
from jax import lax
from jax.experimental import pallas as pl
from jax.experimental.pallas import tpu as pltpu
from jax.experimental.pallas import tpu_sc as plsc

EPS = 1e-5
_NEG = -3.4e38
BT = 256
CAP = 36864
NBLK = CAP // BT


def _normalize(x):
    m = jnp.mean(x, axis=-1, keepdims=True)
    v = jnp.mean((x - m) ** 2, axis=-1, keepdims=True)
    return (x - m) * jax.lax.rsqrt(v + EPS)


def _gelu(x):
    return x * 0.5 * (1.0 + jax.lax.erf(x * (1.0 / math.sqrt(2.0))))


def _router_block(ln_in_g_r, ln_in_b_r, Wr_r, cr_r, x_r,
                  znx_r, logits_r, i1_r, i2_r, w1_r, w2_r,
                  frac_r, prob_r, cnt_r):
    i = pl.program_id(0)
    nsteps = pl.num_programs(0)
    E = Wr_r.shape[1]

    x = x_r[...]
    xn = _normalize(x) * ln_in_g_r[...] + ln_in_b_r[...]
    znx = _normalize(xn)
    znx_r[...] = znx

    logits = jnp.dot(znx, Wr_r[...], preferred_element_type=jnp.float32)
    logits = logits + cr_r[...]
    logits_r[...] = logits

    iota = jax.lax.broadcasted_iota(jnp.int32, logits.shape, 1)
    v1 = jnp.max(logits, axis=-1, keepdims=True)
    i1 = jnp.min(jnp.where(logits == v1, iota, E), axis=-1, keepdims=True)
    masked = jnp.where(iota == i1, _NEG, logits)
    v2 = jnp.max(masked, axis=-1, keepdims=True)
    i2 = jnp.min(jnp.where(masked == v2, iota, E), axis=-1, keepdims=True)
    t = jnp.exp(v2 - v1)
    w1 = 1.0 / (1.0 + t)
    w2 = t / (1.0 + t)

    i1_r[...] = i1[:, 0][None, None, :]
    i2_r[...] = i2[:, 0][None, None, :]
    w1_r[...] = w1[:, 0][None, None, :]
    w2_r[...] = w2[:, 0][None, None, :]

    routed = ((iota == i1).astype(jnp.float32)
              + (iota == i2).astype(jnp.float32))
    sm = jnp.exp(logits - v1)
    sm = sm / jnp.sum(sm, axis=-1, keepdims=True)

    @pl.when(i == 0)
    def _():
        frac_r[...] = jnp.zeros_like(frac_r)
        prob_r[...] = jnp.zeros_like(prob_r)
        cnt_r[...] = jnp.zeros_like(cnt_r)

    frac_r[...] += jnp.sum(routed, axis=0, keepdims=True)
    prob_r[...] += jnp.sum(sm, axis=0, keepdims=True)
    cnt_r[...] += jnp.sum(routed, axis=0, keepdims=True)

    @pl.when(i == nsteps - 1)
    def _():
        n_total = nsteps * x.shape[0]
        frac_r[...] *= 1.0 / n_total
        prob_r[...] *= 1.0 / n_total


def _pos_block(cnt_r, i1_r, i2_r, pos0_r, pos1_r, blk_r, carry_r, off_r):
    i = pl.program_id(0)
    B = i1_r.shape[2]
    E = cnt_r.shape[1]

    @pl.when(i == 0)
    def _():
        cnt = cnt_r[...]
        capac = jnp.ceil(cnt * (1.0 / BT)) * float(BT)
        tri = (jax.lax.broadcasted_iota(jnp.int32, (E, E), 0)
               < jax.lax.broadcasted_iota(jnp.int32, (E, E), 1)
               ).astype(jnp.float32)
        off = jnp.dot(capac, tri, preferred_element_type=jnp.float32)
        off_r[...] = off
        carry_r[...] = jnp.zeros_like(carry_r)
        bpos = jax.lax.broadcasted_iota(
            jnp.int32, (1, NBLK), 1).astype(jnp.float32) * float(BT)
        be = jnp.zeros((1, NBLK), jnp.float32)
        for e in range(1, E):
            off_e = off[:, e:e + 1]
            be = be + (bpos >= off_e).astype(jnp.float32)
        blk_r[...] = be.astype(jnp.int32)

    off = off_r[...]
    tril = (jax.lax.broadcasted_iota(jnp.int32, (B, B), 0)
            >= jax.lax.broadcasted_iota(jnp.int32, (B, B), 1)
            ).astype(jnp.bfloat16)
    iota_e = jax.lax.broadcasted_iota(jnp.int32, (B, E), 1)

    for keys_r, pos_r in ((i1_r, pos0_r), (i2_r, pos1_r)):
        keys = keys_r[0, 0, :]
        mask = (keys[:, None] == iota_e).astype(jnp.float32)
        incl = jnp.dot(tril, mask.astype(jnp.bfloat16),
                       preferred_element_type=jnp.float32)
        base = off + carry_r[...]
        slot = jnp.sum(mask * (base + incl - 1.0), axis=-1)
        pos_r[...] = slot.astype(jnp.int32)[None, None, :]
        carry_r[...] += jnp.sum(mask, axis=0, keepdims=True)


def _expert_block(be_r, zp_r, W1_r, c1_r, W2_r, c2_r, W3_r, c3_r, out_r):
    e = be_r[pl.program_id(0)]
    zb = zp_r[...].astype(jnp.bfloat16)
    h1 = jnp.dot(zb, W1_r[e], preferred_element_type=jnp.float32)
    h1 = _gelu(h1 + c1_r[e])
    n1 = _normalize(h1).astype(jnp.bfloat16)
    h2 = jnp.dot(n1, W2_r[e], preferred_element_type=jnp.float32)
    h2 = _gelu(h2 + c2_r[e])
    n2 = _normalize(h2)
    out_e = jnp.sum(n2 * W3_r[e], axis=-1) + c3_r[e, 0]
    out_r[...] = out_e[:, None]


def _make_sc_kernels(N, D):
    mesh = plsc.VectorSubcoreMesh(core_axis_name="c", subcore_axis_name="s")
    NW = 32
    A_PER_W = 2 * N // NW
    S_PER_W = CAP // NW
    T_PER_W = N // NW

    @functools.partial(
        pl.kernel, mesh=mesh,
        out_type=jax.ShapeDtypeStruct((CAP,), jnp.int32),
        scratch_types=[
            pltpu.VMEM((8, 128), jnp.int32),
            pltpu.VMEM((8, 128), jnp.int32),
            pltpu.SemaphoreType.DMA,
        ],
    )
    def k_scatter(poscat, tokperm, posv, tokv, sem):
        wid = lax.axis_index("s") * 2 + lax.axis_index("c")
        base = wid * A_PER_W
        for j in range(8):
            pltpu.sync_copy(poscat.at[pl.ds(base + j * 128, 128)],
                            posv.at[j])
            for i in range(8):
                a16 = (base + j * 128 + i * 16
                       + jax.lax.iota(jnp.int32, 16))
                tokv[j, pl.ds(i * 16, 16)] = a16 & (N - 1)
        copies = []
        for j in range(8):
            copies.append(pltpu.async_copy(
                tokv.at[j], tokperm.at[posv.at[j]], sem))
        for c in copies:
            c.wait()

    @functools.partial(
        pl.kernel, mesh=mesh,
        out_type=jax.ShapeDtypeStruct((CAP, D), jnp.float32),
        scratch_types=[
            pltpu.VMEM((18, 64), jnp.int32),
            pltpu.VMEM((64, D), jnp.float32),
            pltpu.VMEM((64, D), jnp.float32),
            pltpu.SemaphoreType.DMA,
            pltpu.SemaphoreType.DMA,
            pltpu.SemaphoreType.DMA,
        ],
    )
    def k_gather(tokperm, znx, znx_perm, idxv, rows0, rows1,
                 gsem, wsem0, wsem1):
        wid = lax.axis_index("s") * 2 + lax.axis_index("c")
        base = wid * S_PER_W
        for j in range(18):
            pltpu.sync_copy(tokperm.at[pl.ds(base + j * 64, 64)],
                            idxv.at[j])
        for j in range(18):
            for i in range(4):
                v = idxv[j, pl.ds(i * 16, 16)]
                idxv[j, pl.ds(i * 16, 16)] = v & (N - 1)
        bufs = (rows0, rows1)
        wsems = (wsem0, wsem1)
        g = [None, None]
        w = [None, None]
        for j in range(18):
            b = j % 2
            if w[b] is not None:
                w[b].wait()
            g[b] = pltpu.async_copy(znx.at[idxv.at[j]], bufs[b], gsem)
            if j >= 1:
                bp = (j - 1) % 2
                g[bp].wait()
                w[bp] = pltpu.async_copy(
                    bufs[bp],
                    znx_perm.at[pl.ds(base + (j - 1) * 64, 64)],
                    wsems[bp])
        g[1].wait()
        pltpu.sync_copy(rows1, znx_perm.at[pl.ds(base + 17 * 64, 64)])
        w[0].wait()

    @functools.partial(
        pl.kernel, mesh=mesh,
        out_type=jax.ShapeDtypeStruct((N,), jnp.float32),
        scratch_types=[
            pltpu.VMEM((4, 128), jnp.int32),
            pltpu.VMEM((4, 128), jnp.int32),
            pltpu.VMEM((4, 128), jnp.float32),
            pltpu.VMEM((4, 128), jnp.float32),
            pltpu.VMEM((4, 128), jnp.float32),
            pltpu.VMEM((4, 128), jnp.float32),
            pltpu.VMEM((4, 128), jnp.float32),
            pltpu.SemaphoreType.DMA,
        ],
    )
    def k_combine(pos0, pos1, w1, w2, outperm, final,
                  p0, p1, wv0, wv1, g0, g1, fin, sem):
        wid = lax.axis_index("s") * 2 + lax.axis_index("c")
        base = wid * T_PER_W
        for j in range(4):
            pltpu.sync_copy(pos0.at[pl.ds(base + j * 128, 128)], p0.at[j])
            pltpu.sync_copy(pos1.at[pl.ds(base + j * 128, 128)], p1.at[j])
            pltpu.sync_copy(w1.at[pl.ds(base + j * 128, 128)], wv0.at[j])
            pltpu.sync_copy(w2.at[pl.ds(base + j * 128, 128)], wv1.at[j])
        for j in range(4):
            pltpu.async_copy(outperm.at[p0.at[j]], g0.at[j], sem).wait()
            pltpu.async_copy(outperm.at[p1.at[j]], g1.at[j], sem).wait()
        for j in range(4):
            for r in range(8):
                sl = pl.ds(r * 16, 16)
                fin[j, sl] = g0[j, sl] * wv0[j, sl] + g1[j, sl] * wv1[j, sl]
        for j in range(4):
            pltpu.sync_copy(fin.at[j],
                            final.at[pl.ds(base + j * 128, 128)])

    return k_scatter, k_gather, k_combine


def kernel(x, ln_in_g, ln_in_b, ln_r_g, ln_r_b, W_r, b_r, e_ln1_g, e_ln1_b,
           e_W1, e_b1, e_ln2_g, e_ln2_b, e_W2, e_b2, e_ln3_g, e_ln3_b,
           e_W3, e_b3):
    N, D = x.shape
    E = e_W1.shape[0]
    H = e_W1.shape[2]
    H2 = e_W2.shape[2]
    B = 512
    grid = (N // B,)
    NB = N // B

    Wr_f = ln_r_g[:, None] * W_r
    cr = ln_r_b @ W_r + b_r
    W1_f = (e_ln1_g[:, :, None] * e_W1).astype(jnp.bfloat16)
    c1 = jnp.einsum("ed,edh->eh", e_ln1_b, e_W1) + e_b1
    W2_f = (e_ln2_g[:, :, None] * e_W2).astype(jnp.bfloat16)
    c2 = jnp.einsum("eh,ehk->ek", e_ln2_b, e_W2) + e_b2
    W3_f = e_ln3_g * e_W3[:, :, 0]
    c3 = (jnp.sum(e_ln3_b * e_W3[:, :, 0], axis=-1, keepdims=True) + e_b3)

    full = lambda *s: pl.BlockSpec(s, lambda i: (0,) * len(s))

    a_out = pl.pallas_call(
        _router_block,
        grid=grid,
        in_specs=[full(1, D), full(1, D), full(D, E), full(1, E),
                  pl.BlockSpec((B, D), lambda i: (i, 0))],
        out_specs=[
            pl.BlockSpec((B, D), lambda i: (i, 0)),
            pl.BlockSpec((B, E), lambda i: (i, 0)),
            pl.BlockSpec((1, 1, B), lambda i: (i, 0, 0)),
            pl.BlockSpec((1, 1, B), lambda i: (i, 0, 0)),
            pl.BlockSpec((1, 1, B), lambda i: (i, 0, 0)),
            pl.BlockSpec((1, 1, B), lambda i: (i, 0, 0)),
            pl.BlockSpec((1, E), lambda i: (0, 0)),
            pl.BlockSpec((1, E), lambda i: (0, 0)),
            pl.BlockSpec((1, E), lambda i: (0, 0)),
        ],
        out_shape=(
            jax.ShapeDtypeStruct((N, D), jnp.float32),
            jax.ShapeDtypeStruct((N, E), jnp.float32),
            jax.ShapeDtypeStruct((NB, 1, B), jnp.int32),
            jax.ShapeDtypeStruct((NB, 1, B), jnp.int32),
            jax.ShapeDtypeStruct((NB, 1, B), jnp.float32),
            jax.ShapeDtypeStruct((NB, 1, B), jnp.float32),
            jax.ShapeDtypeStruct((1, E), jnp.float32),
            jax.ShapeDtypeStruct((1, E), jnp.float32),
            jax.ShapeDtypeStruct((1, E), jnp.float32),
        ),
    )(ln_in_g[None], ln_in_b[None], Wr_f, cr[None], x)
    znx, logits, i1b, i2b, w1b, w2b, frac, prob, cnt = a_out

    pos0b, pos1b, blk = pl.pallas_call(
        _pos_block,
        grid=grid,
        in_specs=[full(1, E),
                  pl.BlockSpec((1, 1, B), lambda i: (i, 0, 0)),
                  pl.BlockSpec((1, 1, B), lambda i: (i, 0, 0))],
        out_specs=[
            pl.BlockSpec((1, 1, B), lambda i: (i, 0, 0)),
            pl.BlockSpec((1, 1, B), lambda i: (i, 0, 0)),
            pl.BlockSpec((1, NBLK), lambda i: (0, 0)),
        ],
        out_shape=(
            jax.ShapeDtypeStruct((NB, 1, B), jnp.int32),
            jax.ShapeDtypeStruct((NB, 1, B), jnp.int32),
            jax.ShapeDtypeStruct((1, NBLK), jnp.int32),
        ),
        scratch_shapes=[pltpu.VMEM((1, E), jnp.float32),
                        pltpu.VMEM((1, E), jnp.float32)],
    )(cnt, i1b, i2b)

    pos0 = pos0b.reshape(N)
    pos1 = pos1b.reshape(N)
    poscat = jnp.concatenate([pos0, pos1], axis=0)

    k_scatter, k_gather, k_combine = _make_sc_kernels(N, D)
    tokperm = k_scatter(poscat)
    znx_perm = k_gather(tokperm, znx)

    outperm = pl.pallas_call(
        _expert_block,
        grid_spec=pltpu.PrefetchScalarGridSpec(
            num_scalar_prefetch=1,
            grid=(NBLK,),
            in_specs=[
                pl.BlockSpec((BT, D), lambda i, s: (i, 0)),
                pl.BlockSpec((E, D, H), lambda i, s: (0, 0, 0)),
                pl.BlockSpec((E, 1, H), lambda i, s: (0, 0, 0)),
                pl.BlockSpec((E, H, H2), lambda i, s: (0, 0, 0)),
                pl.BlockSpec((E, 1, H2), lambda i, s: (0, 0, 0)),
                pl.BlockSpec((E, 1, H2), lambda i, s: (0, 0, 0)),
                pl.BlockSpec((E, 1), lambda i, s: (0, 0)),
            ],
            out_specs=pl.BlockSpec((BT, 1), lambda i, s: (i, 0)),
        ),
        out_shape=jax.ShapeDtypeStruct((CAP, 1), jnp.float32),
    )(blk.reshape(NBLK), znx_perm, W1_f, c1[:, None], W2_f, c2[:, None],
      W3_f[:, None], c3)

    final = k_combine(pos0, pos1, w1b.reshape(N), w2b.reshape(N),
                      outperm.reshape(CAP))

    return (final[:, None], frac[0], prob[0], logits)
